# baseline (speedup 1.0000x reference)
import functools

import jax
import jax.numpy as jnp
from jax import lax
from jax.experimental import pallas as pl
from jax.experimental.pallas import tpu as pltpu
from jax.experimental.pallas import tpu_sc as plsc

_RB = 400
_CHUNK = 128
_NBUF = 4

_SC_PARAMS = pltpu.CompilerParams(
    needs_layout_passes=False, use_tc_tiling_on_sc=False
)


def _popcount_u32(v):
    m1 = jnp.uint32(0x55555555)
    m2 = jnp.uint32(0x33333333)
    m4 = jnp.uint32(0x0F0F0F0F)
    h01 = jnp.uint32(0x01010101)
    v = v - ((v >> jnp.uint32(1)) & m1)
    v = (v & m2) + ((v >> jnp.uint32(2)) & m2)
    v = (v + (v >> jnp.uint32(4))) & m4
    return (v * h01) >> jnp.uint32(24)


def _prep_body(x_ref, w1_ref, bits_ref, nnz_ref, xw_ref):
    xb = x_ref[...]
    nz = (xb != 0.0).astype(jnp.int32)
    lane = lax.broadcasted_iota(jnp.int32, xb.shape, 1)
    shifted = jnp.left_shift(nz, lane % 32)
    cols = [
        jnp.sum(shifted[:, 32 * w : 32 * w + 32], axis=1, keepdims=True)
        for w in range(4)
    ]
    bits_ref[...] = jnp.concatenate(cols, axis=1)
    nnz_ref[...] = jnp.sum(nz, axis=1, keepdims=True)
    xw_ref[...] = jnp.dot(xb, w1_ref[...], preferred_element_type=jnp.float32)


def _prep(x, W1):
    N, D = x.shape
    H = W1.shape[1]
    grid = N // _RB
    return pl.pallas_call(
        _prep_body,
        grid=(grid,),
        in_specs=[
            pl.BlockSpec((_RB, D), lambda i: (i, 0)),
            pl.BlockSpec((D, H), lambda i: (0, 0)),
        ],
        out_specs=[
            pl.BlockSpec((_RB, 4), lambda i: (i, 0)),
            pl.BlockSpec((_RB, 1), lambda i: (i, 0)),
            pl.BlockSpec((_RB, H), lambda i: (i, 0)),
        ],
        out_shape=[
            jax.ShapeDtypeStruct((N, 4), jnp.int32),
            jax.ShapeDtypeStruct((N, 1), jnp.int32),
            jax.ShapeDtypeStruct((N, H), jnp.float32),
        ],
    )(x, W1)


def _dinv_body(degp_ref, xw_ref, dinv_ref, g_ref):
    deg = degp_ref[0, :, 0:1] + degp_ref[1, :, 0:1] + 1.0
    dinv = lax.rsqrt(deg)
    dinv_ref[...] = dinv
    g_ref[...] = dinv * xw_ref[...]


def _dinv_g(degp, xw1):
    N, H = xw1.shape
    grid = N // _RB
    return pl.pallas_call(
        _dinv_body,
        grid=(grid,),
        in_specs=[
            pl.BlockSpec((2, _RB, 16), lambda i: (0, i, 0)),
            pl.BlockSpec((_RB, H), lambda i: (i, 0)),
        ],
        out_specs=[
            pl.BlockSpec((_RB, 1), lambda i: (i, 0)),
            pl.BlockSpec((_RB, H), lambda i: (i, 0)),
        ],
        out_shape=[
            jax.ShapeDtypeStruct((N, 1), jnp.float32),
            jax.ShapeDtypeStruct((N, H), jnp.float32),
        ],
    )(degp, xw1)


def _make_fuse1_body(c0, c1):
    def body(p_ref, g_ref, dinv_ref, b_ref, w_ref, out_ref):
        dinv = dinv_ref[...]
        t = p_ref[0] + p_ref[1] + g_ref[...]
        h = jnp.maximum(dinv * t[:, c0:c1] + b_ref[...], 0.0)
        out_ref[...] = dinv * jnp.dot(
            h, w_ref[...], preferred_element_type=jnp.float32
        )

    return body


def _fuse1(p, g, dinv, b, W, c0=None, c1=None):
    N, H = g.shape
    if c0 is None:
        c0, c1 = 0, H
    Hb = c1 - c0
    H2 = W.shape[1]
    grid = N // _RB
    return pl.pallas_call(
        _make_fuse1_body(c0, c1),
        grid=(grid,),
        in_specs=[
            pl.BlockSpec((2, _RB, H), lambda i: (0, i, 0)),
            pl.BlockSpec((_RB, H), lambda i: (i, 0)),
            pl.BlockSpec((_RB, 1), lambda i: (i, 0)),
            pl.BlockSpec((1, Hb), lambda i: (0, 0)),
            pl.BlockSpec((Hb, H2), lambda i: (0, 0)),
        ],
        out_specs=pl.BlockSpec((_RB, H2), lambda i: (i, 0)),
        out_shape=jax.ShapeDtypeStruct((N, H2), jnp.float32),
    )(p, g, dinv, b.reshape(1, Hb), W)


def _make_fuse_out_body(c0, c1):
    def body(p_ref, g_ref, dinv_ref, b_ref, out_ref):
        t = p_ref[0] + p_ref[1] + g_ref[...]
        out_ref[...] = jnp.maximum(
            dinv_ref[...] * t[:, c0:c1] + b_ref[...], 0.0
        )

    return body


def _fuse_out(p, g, dinv, b, c0=None, c1=None):
    N, H = g.shape
    if c0 is None:
        c0, c1 = 0, H
    Hb = c1 - c0
    grid = N // _RB
    return pl.pallas_call(
        _make_fuse_out_body(c0, c1),
        grid=(grid,),
        in_specs=[
            pl.BlockSpec((2, _RB, H), lambda i: (0, i, 0)),
            pl.BlockSpec((_RB, H), lambda i: (i, 0)),
            pl.BlockSpec((_RB, 1), lambda i: (i, 0)),
            pl.BlockSpec((1, Hb), lambda i: (0, 0)),
        ],
        out_specs=pl.BlockSpec((_RB, Hb), lambda i: (i, 0)),
        out_shape=jax.ShapeDtypeStruct((N, Hb), jnp.float32),
    )(p, g, dinv, b.reshape(1, Hb))


def _ahat_body(a_ref, b_ref, o_ref):
    o_ref[...] = lax.dot_general(
        a_ref[...],
        b_ref[...],
        (((1,), (1,)), ((), ())),
        preferred_element_type=jnp.float32,
    )


def _ahat(s_dec, bm=1024, bn=1024):
    N, H = s_dec.shape
    gm = pl.cdiv(N, bm)
    gn = pl.cdiv(N, bn)
    return pl.pallas_call(
        _ahat_body,
        grid=(gm, gn),
        in_specs=[
            pl.BlockSpec((bm, H), lambda i, j: (i, 0)),
            pl.BlockSpec((bn, H), lambda i, j: (j, 0)),
        ],
        out_specs=pl.BlockSpec((bm, bn), lambda i, j: (i, j)),
        out_shape=jax.ShapeDtypeStruct((N, N), jnp.float32),
    )(s_dec, s_dec)


def _make_jaccard(N, E, NC, NS, NP, RPT, CPT, NCH):
    mesh = plsc.VectorSubcoreMesh(core_axis_name="c", subcore_axis_name="s")

    @functools.partial(
        pl.kernel,
        out_type=[
            jax.ShapeDtypeStruct((NCH, _CHUNK), jnp.int32),
            jax.ShapeDtypeStruct((NC, NP, 16), jnp.float32),
        ],
        mesh=mesh,
        compiler_params=_SC_PARAMS,
        scratch_types=[
            pltpu.VMEM((N * 4,), jnp.int32),
            pltpu.VMEM((N,), jnp.int32),
            pltpu.VMEM((_CHUNK,), jnp.int32),
            pltpu.VMEM((_CHUNK,), jnp.int32),
            pltpu.VMEM((_CHUNK,), jnp.int32),
            pltpu.VMEM((_CHUNK, 16), jnp.float32),
            pltpu.VMEM_SHARED((NP, 16), jnp.float32),
        ],
    )
    def jac(
        bits_h,
        nnz_h,
        src_h,
        dst_h,
        ones_h,
        z16_h,
        dstp_h,
        degp_h,
        bits_v,
        nnz_v,
        ids_v,
        idd_v,
        dstp_v,
        ones_v,
        deg_sh,
    ):
        c = lax.axis_index("c")
        s = lax.axis_index("s")
        wid = s * NC + c
        pltpu.sync_copy(bits_h, bits_v)
        pltpu.sync_copy(nnz_h, nnz_v)
        pltpu.sync_copy(ones_h, ones_v)
        pltpu.sync_copy(z16_h, deg_sh.at[pl.ds(s * RPT, RPT)])
        plsc.subcore_barrier()
        lane = jnp.arange(16, dtype=jnp.int32)

        @pl.loop(0, CPT)
        def _(i):
            cid = wid * CPT + i
            pltpu.sync_copy(src_h.at[cid], ids_v)
            pltpu.sync_copy(dst_h.at[cid], idd_v)
            ebase = cid * _CHUNK
            for k in range(_CHUNK // 16):
                sv = ids_v[pl.ds(k * 16, 16)]
                dv = idd_v[pl.ds(k * 16, 16)]
                inter = jnp.zeros((16,), jnp.uint32)
                for w in range(4):
                    a = plsc.load_gather(bits_v, [sv * 4 + w])
                    b = plsc.load_gather(bits_v, [dv * 4 + w])
                    inter = inter + _popcount_u32(plsc.bitcast(a & b, jnp.uint32))
                nsum = plsc.load_gather(nnz_v, [sv]) + plsc.load_gather(
                    nnz_v, [dv]
                )
                eidv = ebase + k * 16 + lane
                keep = (inter.astype(jnp.int32) * 51) > nsum
                keep = keep & (eidv < E)
                dummy = (eidv & 63) + (N + 1)
                dstp_v[pl.ds(k * 16, 16)] = jnp.where(keep, dv, dummy)
            pltpu.sync_copy(dstp_v, dstp_h.at[cid])
            pltpu.sync_copy(ones_v, deg_sh.at[dstp_v], add=True)

        plsc.subcore_barrier()
        pltpu.sync_copy(
            deg_sh.at[pl.ds(s * RPT, RPT)], degp_h.at[c, pl.ds(s * RPT, RPT)]
        )

    return jac


def _make_agg(N, H, NC, NS, NP, RPT, CPT, NCH, nbuf=_NBUF):
    mesh = plsc.VectorSubcoreMesh(core_axis_name="c", subcore_axis_name="s")

    @functools.partial(
        pl.kernel,
        out_type=jax.ShapeDtypeStruct((NC, NP, H), jnp.float32),
        mesh=mesh,
        compiler_params=_SC_PARAMS,
        scratch_types=[
            pltpu.VMEM((CPT, _CHUNK), jnp.int32),
            pltpu.VMEM((CPT, _CHUNK), jnp.int32),
            [pltpu.VMEM((_CHUNK, H), jnp.float32) for _ in range(nbuf)],
            pltpu.VMEM_SHARED((NP, H), jnp.float32),
            [pltpu.SemaphoreType.DMA for _ in range(nbuf)],
            [pltpu.SemaphoreType.DMA for _ in range(nbuf)],
        ],
    )
    def agg(g_h, src_h, dstp_h, z_h, accp_h, ids_v, idd_v, rows, acc_sh, gsem, ssem):
        c = lax.axis_index("c")
        s = lax.axis_index("s")
        wid = s * NC + c
        pltpu.sync_copy(src_h.at[pl.ds(wid * CPT, CPT)], ids_v)
        pltpu.sync_copy(dstp_h.at[pl.ds(wid * CPT, CPT)], idd_v)
        pltpu.sync_copy(z_h, acc_sh.at[pl.ds(s * RPT, RPT)])
        plsc.subcore_barrier()

        gd = {}
        sd = {}
        for b in range(nbuf):
            gd[b] = pltpu.async_copy(g_h.at[ids_v.at[b]], rows[b], gsem[b])
        for i in range(CPT):
            b = i % nbuf
            gd[b].wait()
            sd[b] = pltpu.async_copy(
                rows[b], acc_sh.at[idd_v.at[i]], ssem[b], add=True
            )
            j = i + 2
            if nbuf <= j < CPT:
                bj = j % nbuf
                sd[bj].wait()
                gd[bj] = pltpu.async_copy(
                    g_h.at[ids_v.at[j]], rows[bj], gsem[bj]
                )
        for b in range(nbuf):
            sd[b].wait()
        plsc.subcore_barrier()
        pltpu.sync_copy(
            acc_sh.at[pl.ds(s * RPT, RPT)], accp_h.at[c, pl.ds(s * RPT, RPT)]
        )

    return agg


def kernel(x, edge_index, W1, b1, W2, b2, W3, b3, W4, b4, W5, b5):
    N, D = x.shape
    E = edge_index.shape[1]
    H = W1.shape[1]

    info = plsc.get_sparse_core_info()
    NC, NS = info.num_cores, info.num_subcores
    NW = NC * NS
    RPT = 8 * pl.cdiv(pl.cdiv(N + 1, NS), 8)
    NP = RPT * NS
    CPT = pl.cdiv(E, _CHUNK * NW)
    NCH = CPT * NW
    E2 = NCH * _CHUNK

    src = edge_index[0]
    dst = edge_index[1]
    pad = jnp.zeros((E2 - E,), jnp.int32)
    src2 = jnp.concatenate([src, pad]).reshape(NCH, _CHUNK)
    dst2 = jnp.concatenate([dst, pad]).reshape(NCH, _CHUNK)

    ones16 = jnp.ones((_CHUNK, 16), jnp.float32)
    z16 = jnp.zeros((RPT, 16), jnp.float32)
    zH = jnp.zeros((RPT, H), jnp.float32)
    zD = jnp.zeros((RPT, D), jnp.float32)
    W35 = jnp.concatenate([W3, W5], axis=1)

    jac = _make_jaccard(N, E, NC, NS, NP, RPT, CPT, NCH)
    aggH = _make_agg(N, H, NC, NS, NP, RPT, CPT, NCH)
    aggD = _make_agg(N, D, NC, NS, NP, RPT, CPT, NCH, nbuf=2)

    bits, nnz, xw1 = _prep(x, W1)
    dstp, degp = jac(bits.reshape(N * 4), nnz.reshape(N), src2, dst2, ones16, z16)
    dinv, g1 = _dinv_g(degp, xw1)

    acc1 = aggH(g1, src2, dstp, zH)
    g2 = _fuse1(acc1, g1, dinv, b1, W2)

    acc2 = aggH(g2, src2, dstp, zH)
    g35 = _fuse1(acc2, g2, dinv, b2, W35)

    acc35 = aggD(g35, src2, dstp, zD)
    s_dec = _fuse_out(acc35, g35, dinv, b5, H, 2 * H)
    g4 = _fuse1(acc35, g35, dinv, b3, W4, 0, H)

    acc4 = aggD(g4, src2, dstp, zD)
    x_hat = _fuse_out(acc4, g4, dinv, b4)

    A_hat = _ahat(s_dec)
    return (A_hat, x_hat)


# speedup vs baseline: 1.2158x; 1.2158x over previous
import functools

import jax
import jax.numpy as jnp
from jax import lax
from jax.experimental import pallas as pl
from jax.experimental.pallas import tpu as pltpu
from jax.experimental.pallas import tpu_sc as plsc

_RB = 400
_CHUNK = 128
_NBUF = 4

_SC_PARAMS = pltpu.CompilerParams(
    needs_layout_passes=False, use_tc_tiling_on_sc=False
)


def _popcount_u32(v):
    m1 = jnp.uint32(0x55555555)
    m2 = jnp.uint32(0x33333333)
    m4 = jnp.uint32(0x0F0F0F0F)
    h01 = jnp.uint32(0x01010101)
    v = v - ((v >> jnp.uint32(1)) & m1)
    v = (v & m2) + ((v >> jnp.uint32(2)) & m2)
    v = (v + (v >> jnp.uint32(4))) & m4
    return (v * h01) >> jnp.uint32(24)


def _prep_body(x_ref, w1_ref, bits_ref, nnz_ref, xw_ref):
    xb = x_ref[...]
    nz = (xb != 0.0).astype(jnp.int32)
    lane = lax.broadcasted_iota(jnp.int32, xb.shape, 1)
    shifted = jnp.left_shift(nz, lane % 32)
    cols = [
        jnp.sum(shifted[:, 32 * w : 32 * w + 32], axis=1, keepdims=True)
        for w in range(4)
    ]
    bits_ref[...] = jnp.concatenate(cols, axis=1)
    nnz_ref[...] = jnp.sum(nz, axis=1, keepdims=True)
    xw_ref[...] = jnp.dot(xb, w1_ref[...], preferred_element_type=jnp.float32)


def _prep(x, W1):
    N, D = x.shape
    H = W1.shape[1]
    grid = N // _RB
    return pl.pallas_call(
        _prep_body,
        grid=(grid,),
        in_specs=[
            pl.BlockSpec((_RB, D), lambda i: (i, 0)),
            pl.BlockSpec((D, H), lambda i: (0, 0)),
        ],
        out_specs=[
            pl.BlockSpec((_RB, 4), lambda i: (i, 0)),
            pl.BlockSpec((_RB, 1), lambda i: (i, 0)),
            pl.BlockSpec((_RB, H), lambda i: (i, 0)),
        ],
        out_shape=[
            jax.ShapeDtypeStruct((N, 4), jnp.int32),
            jax.ShapeDtypeStruct((N, 1), jnp.int32),
            jax.ShapeDtypeStruct((N, H), jnp.float32),
        ],
    )(x, W1)


def _write_slabs(res, gs_ref):
    h = res.shape[1] // 2
    gs_ref[0] = res[:, :h]
    gs_ref[1] = res[:, h:]


def _dinv_body(degp_ref, xw_ref, dinv_ref, gs_ref):
    deg = degp_ref[0, :, 0:1] + degp_ref[1, :, 0:1] + 1.0
    dinv = lax.rsqrt(deg)
    dinv_ref[...] = dinv
    _write_slabs(dinv * xw_ref[...], gs_ref)


def _dinv_g(degp, xw1):
    N, H = xw1.shape
    Hh = H // 2
    grid = N // _RB
    return pl.pallas_call(
        _dinv_body,
        grid=(grid,),
        in_specs=[
            pl.BlockSpec((2, _RB, 16), lambda i: (0, i, 0)),
            pl.BlockSpec((_RB, H), lambda i: (i, 0)),
        ],
        out_specs=[
            pl.BlockSpec((_RB, 1), lambda i: (i, 0)),
            pl.BlockSpec((2, _RB, Hh), lambda i: (0, i, 0)),
        ],
        out_shape=[
            jax.ShapeDtypeStruct((N, 1), jnp.float32),
            jax.ShapeDtypeStruct((2, N, Hh), jnp.float32),
        ],
    )(degp, xw1)


def _make_fuse1_body(c0, c1):
    def body(p_ref, g_ref, dinv_ref, b_ref, w_ref, out_ref):
        dinv = dinv_ref[...]
        t = jnp.concatenate(
            [p_ref[0] + g_ref[0], p_ref[1] + g_ref[1]], axis=1
        )
        h = jnp.maximum(dinv * t[:, c0:c1] + b_ref[...], 0.0)
        _write_slabs(
            dinv * jnp.dot(h, w_ref[...], preferred_element_type=jnp.float32),
            out_ref,
        )

    return body


def _fuse1(p, g, dinv, b, W, c0=None, c1=None):
    _, N, Hh = g.shape
    if c0 is None:
        c0, c1 = 0, 2 * Hh
    Hb = c1 - c0
    H2 = W.shape[1]
    grid = N // _RB
    return pl.pallas_call(
        _make_fuse1_body(c0, c1),
        grid=(grid,),
        in_specs=[
            pl.BlockSpec((2, _RB, Hh), lambda i: (0, i, 0)),
            pl.BlockSpec((2, _RB, Hh), lambda i: (0, i, 0)),
            pl.BlockSpec((_RB, 1), lambda i: (i, 0)),
            pl.BlockSpec((1, Hb), lambda i: (0, 0)),
            pl.BlockSpec((Hb, H2), lambda i: (0, 0)),
        ],
        out_specs=pl.BlockSpec((2, _RB, H2 // 2), lambda i: (0, i, 0)),
        out_shape=jax.ShapeDtypeStruct((2, N, H2 // 2), jnp.float32),
    )(p, g, dinv, b.reshape(1, Hb), W)


def _make_fuse_out_body(c0, c1):
    def body(p_ref, g_ref, dinv_ref, b_ref, out_ref):
        t = jnp.concatenate(
            [p_ref[0] + g_ref[0], p_ref[1] + g_ref[1]], axis=1
        )
        out_ref[...] = jnp.maximum(
            dinv_ref[...] * t[:, c0:c1] + b_ref[...], 0.0
        )

    return body


def _fuse_out(p, g, dinv, b, c0=None, c1=None):
    _, N, Hh = g.shape
    if c0 is None:
        c0, c1 = 0, 2 * Hh
    Hb = c1 - c0
    grid = N // _RB
    return pl.pallas_call(
        _make_fuse_out_body(c0, c1),
        grid=(grid,),
        in_specs=[
            pl.BlockSpec((2, _RB, Hh), lambda i: (0, i, 0)),
            pl.BlockSpec((2, _RB, Hh), lambda i: (0, i, 0)),
            pl.BlockSpec((_RB, 1), lambda i: (i, 0)),
            pl.BlockSpec((1, Hb), lambda i: (0, 0)),
        ],
        out_specs=pl.BlockSpec((_RB, Hb), lambda i: (i, 0)),
        out_shape=jax.ShapeDtypeStruct((N, Hb), jnp.float32),
    )(p, g, dinv, b.reshape(1, Hb))


def _ahat_body(a_ref, b_ref, o_ref):
    o_ref[...] = lax.dot_general(
        a_ref[...],
        b_ref[...],
        (((1,), (1,)), ((), ())),
        preferred_element_type=jnp.float32,
    )


def _ahat(s_dec, bm=1024, bn=1024):
    N, H = s_dec.shape
    gm = pl.cdiv(N, bm)
    gn = pl.cdiv(N, bn)
    return pl.pallas_call(
        _ahat_body,
        grid=(gm, gn),
        in_specs=[
            pl.BlockSpec((bm, H), lambda i, j: (i, 0)),
            pl.BlockSpec((bn, H), lambda i, j: (j, 0)),
        ],
        out_specs=pl.BlockSpec((bm, bn), lambda i, j: (i, j)),
        out_shape=jax.ShapeDtypeStruct((N, N), jnp.float32),
    )(s_dec, s_dec)


def _make_jaccard(N, E, NC, NS, NP, RPT, CPT, NCH):
    mesh = plsc.VectorSubcoreMesh(core_axis_name="c", subcore_axis_name="s")

    @functools.partial(
        pl.kernel,
        out_type=[
            jax.ShapeDtypeStruct((NCH, _CHUNK), jnp.int32),
            jax.ShapeDtypeStruct((NC, NP, 16), jnp.float32),
        ],
        mesh=mesh,
        compiler_params=_SC_PARAMS,
        scratch_types=[
            pltpu.VMEM((N * 4,), jnp.int32),
            pltpu.VMEM((N,), jnp.int32),
            pltpu.VMEM((_CHUNK,), jnp.int32),
            pltpu.VMEM((_CHUNK,), jnp.int32),
            pltpu.VMEM((_CHUNK,), jnp.int32),
            pltpu.VMEM((_CHUNK, 16), jnp.float32),
            pltpu.VMEM_SHARED((NP, 16), jnp.float32),
        ],
    )
    def jac(
        bits_h,
        nnz_h,
        src_h,
        dst_h,
        ones_h,
        z16_h,
        dstp_h,
        degp_h,
        bits_v,
        nnz_v,
        ids_v,
        idd_v,
        dstp_v,
        ones_v,
        deg_sh,
    ):
        c = lax.axis_index("c")
        s = lax.axis_index("s")
        wid = s * NC + c
        pltpu.sync_copy(bits_h, bits_v)
        pltpu.sync_copy(nnz_h, nnz_v)
        pltpu.sync_copy(ones_h, ones_v)
        pltpu.sync_copy(z16_h, deg_sh.at[pl.ds(s * RPT, RPT)])
        plsc.subcore_barrier()
        lane = jnp.arange(16, dtype=jnp.int32)

        @pl.loop(0, CPT)
        def _(i):
            cid = wid * CPT + i
            pltpu.sync_copy(src_h.at[cid], ids_v)
            pltpu.sync_copy(dst_h.at[cid], idd_v)
            ebase = cid * _CHUNK
            for k in range(_CHUNK // 16):
                sv = ids_v[pl.ds(k * 16, 16)]
                dv = idd_v[pl.ds(k * 16, 16)]
                inter = jnp.zeros((16,), jnp.uint32)
                for w in range(4):
                    a = plsc.load_gather(bits_v, [sv * 4 + w])
                    b = plsc.load_gather(bits_v, [dv * 4 + w])
                    inter = inter + _popcount_u32(plsc.bitcast(a & b, jnp.uint32))
                nsum = plsc.load_gather(nnz_v, [sv]) + plsc.load_gather(
                    nnz_v, [dv]
                )
                eidv = ebase + k * 16 + lane
                keep = (inter.astype(jnp.int32) * 51) > nsum
                keep = keep & (eidv < E)
                dummy = (eidv & 63) + (N + 1)
                dstp_v[pl.ds(k * 16, 16)] = jnp.where(keep, dv, dummy)
            pltpu.sync_copy(dstp_v, dstp_h.at[cid])
            pltpu.sync_copy(ones_v, deg_sh.at[dstp_v], add=True)

        plsc.subcore_barrier()
        pltpu.sync_copy(
            deg_sh.at[pl.ds(s * RPT, RPT)], degp_h.at[c, pl.ds(s * RPT, RPT)]
        )

    return jac


def _make_agg(N, H, NC, NS, NP, RPT, NCH, nbuf=_NBUF):
    Hh = H // 2
    CPT2 = NCH // NS
    mesh = plsc.VectorSubcoreMesh(core_axis_name="c", subcore_axis_name="s")

    @functools.partial(
        pl.kernel,
        out_type=jax.ShapeDtypeStruct((NC, NP, Hh), jnp.float32),
        mesh=mesh,
        compiler_params=_SC_PARAMS,
        scratch_types=[
            pltpu.VMEM((CPT2, _CHUNK), jnp.int32),
            pltpu.VMEM((CPT2, _CHUNK), jnp.int32),
            [pltpu.VMEM((_CHUNK, Hh), jnp.float32) for _ in range(nbuf)],
            pltpu.VMEM_SHARED((NP, Hh), jnp.float32),
            [pltpu.SemaphoreType.DMA for _ in range(nbuf)],
            [pltpu.SemaphoreType.DMA for _ in range(nbuf)],
        ],
    )
    def agg(g2d_h, src3_h, dstp_h, z_h, accp_h, ids_v, idd_v, rows, acc_sh, gsem, ssem):
        c = lax.axis_index("c")
        s = lax.axis_index("s")
        pltpu.sync_copy(src3_h.at[c, pl.ds(s * CPT2, CPT2)], ids_v)
        pltpu.sync_copy(dstp_h.at[pl.ds(s * CPT2, CPT2)], idd_v)
        pltpu.sync_copy(z_h, acc_sh.at[pl.ds(s * RPT, RPT)])
        plsc.subcore_barrier()

        gd = {}
        sd = {}
        for b in range(nbuf):
            gd[b] = pltpu.async_copy(g2d_h.at[ids_v.at[b]], rows[b], gsem[b])
        for i in range(CPT2):
            b = i % nbuf
            gd[b].wait()
            sd[b] = pltpu.async_copy(
                rows[b], acc_sh.at[idd_v.at[i]], ssem[b], add=True
            )
            j = i + 2
            if nbuf <= j < CPT2:
                bj = j % nbuf
                sd[bj].wait()
                gd[bj] = pltpu.async_copy(
                    g2d_h.at[ids_v.at[j]], rows[bj], gsem[bj]
                )
        for b in range(nbuf):
            sd[b].wait()
        plsc.subcore_barrier()
        pltpu.sync_copy(
            acc_sh.at[pl.ds(s * RPT, RPT)], accp_h.at[c, pl.ds(s * RPT, RPT)]
        )

    return agg


def kernel(x, edge_index, W1, b1, W2, b2, W3, b3, W4, b4, W5, b5):
    N, D = x.shape
    E = edge_index.shape[1]
    H = W1.shape[1]

    info = plsc.get_sparse_core_info()
    NC, NS = info.num_cores, info.num_subcores
    NW = NC * NS
    RPT = 8 * pl.cdiv(pl.cdiv(N + 65, NS), 8)
    NP = RPT * NS
    CPT = pl.cdiv(E, _CHUNK * NW)
    NCH = CPT * NW
    E2 = NCH * _CHUNK

    src = edge_index[0]
    dst = edge_index[1]
    pad = jnp.zeros((E2 - E,), jnp.int32)
    src2 = jnp.concatenate([src, pad]).reshape(NCH, _CHUNK)
    dst2 = jnp.concatenate([dst, pad]).reshape(NCH, _CHUNK)
    src3 = jnp.stack([src2, src2 + N])

    ones16 = jnp.ones((_CHUNK, 16), jnp.float32)
    z16 = jnp.zeros((RPT, 16), jnp.float32)
    zH = jnp.zeros((RPT, H // 2), jnp.float32)
    zD = jnp.zeros((RPT, D // 2), jnp.float32)
    W35 = jnp.concatenate([W3, W5], axis=1)

    jac = _make_jaccard(N, E, NC, NS, NP, RPT, CPT, NCH)
    aggH = _make_agg(N, H, NC, NS, NP, RPT, NCH)
    aggD = _make_agg(N, D, NC, NS, NP, RPT, NCH)

    bits, nnz, xw1 = _prep(x, W1)
    dstp, degp = jac(bits.reshape(N * 4), nnz.reshape(N), src2, dst2, ones16, z16)
    dinv, g1 = _dinv_g(degp, xw1)

    acc1 = aggH(g1.reshape(2 * N, H // 2), src3, dstp, zH)
    g2 = _fuse1(acc1, g1, dinv, b1, W2)

    acc2 = aggH(g2.reshape(2 * N, H // 2), src3, dstp, zH)
    g35 = _fuse1(acc2, g2, dinv, b2, W35)

    acc35 = aggD(g35.reshape(2 * N, D // 2), src3, dstp, zD)
    s_dec = _fuse_out(acc35, g35, dinv, b5, H, 2 * H)
    g4 = _fuse1(acc35, g35, dinv, b3, W4, 0, H)

    acc4 = aggD(g4.reshape(2 * N, D // 2), src3, dstp, zD)
    x_hat = _fuse_out(acc4, g4, dinv, b4)

    A_hat = _ahat(s_dec)
    return (A_hat, x_hat)
